# baseline (speedup 1.0000x reference)
import functools

import jax
import jax.numpy as jnp
from jax import lax
from jax.experimental import pallas as pl
from jax.experimental.pallas import tpu as pltpu
from jax.experimental.pallas import tpu_sc as plsc

B = 16384
D = 64
TC = 128
NC = 2
NS = 16
NW = NC * NS
BW = B // NW
L = 16
NG = BW // L
BCH = 128
NBCH = BW // BCH
HALF = BW // 2
NBUF = 2

_mesh = plsc.VectorSubcoreMesh(core_axis_name="c", subcore_axis_name="s")


@functools.partial(
    pl.kernel,
    mesh=_mesh,
    compiler_params=pltpu.CompilerParams(needs_layout_passes=False),
    out_type=jax.ShapeDtypeStruct((B,), jnp.float32),
    scratch_types=[
        pltpu.VMEM((BW,), jnp.int32),
        pltpu.VMEM((BW,), jnp.int32),
        pltpu.VMEM((NBUF, D, TC), jnp.float32),
        pltpu.VMEM((NBUF, D, TC), jnp.float32),
        pltpu.VMEM((HALF, D), jnp.float32),
        pltpu.VMEM((HALF, D), jnp.float32),
        pltpu.VMEM((BW,), jnp.float32),
        pltpu.VMEM((BW,), jnp.float32),
        pltpu.VMEM((BW,), jnp.float32),
        pltpu.SemaphoreType.DMA,
        pltpu.SemaphoreType.DMA,
        pltpu.SemaphoreType.DMA,
    ],
)
def _mf_sc(u_hbm, i_hbm, uet_hbm, iet_hbm, ub_hbm, ib_hbm, out_hbm,
           u_v, i_v, utile, itile, ue2, ie2, ub_v, ib_v, o_v,
           sem0, sem1, bsem):
    wid = lax.axis_index("s") * NC + lax.axis_index("c")
    base = wid * BW

    pltpu.sync_copy(u_hbm.at[pl.ds(base, BW)], u_v)
    pltpu.sync_copy(i_hbm.at[pl.ds(base, BW)], i_v)

    bias_copies = []
    for c in range(NBCH):
        sl = pl.ds(c * BCH, BCH)
        bias_copies.append(pltpu.async_copy(ub_hbm.at[u_v.at[sl]], ub_v.at[sl], bsem))
        bias_copies.append(pltpu.async_copy(ib_hbm.at[i_v.at[sl]], ib_v.at[sl], bsem))

    iota = lax.iota(jnp.int32, L)
    sems = (sem0, sem1)

    def tilecol(vec, l):
        return lax.shift_right_logical(vec[l], 7) * TC

    dcols = [jnp.full((L,), q * L, jnp.int32) + iota for q in range(D // L)]

    def do_pass(half):
        def run_group(g, _):
            uvec = u_v[pl.ds(g * L, L)]
            ivec = i_v[pl.ds(g * L, L)]
            for l in range(L):
                buf = l % NBUF
                sem = sems[buf]
                utc = tilecol(uvec, l)
                itc = tilecol(ivec, l)
                pltpu.async_copy(uet_hbm.at[:, pl.ds(utc, TC)], utile.at[buf], sem)
                pltpu.async_copy(iet_hbm.at[:, pl.ds(itc, TC)], itile.at[buf], sem)
                pltpu.make_async_copy(uet_hbm.at[:, pl.ds(utc, TC)], utile.at[buf], sem).wait()
                pltpu.make_async_copy(iet_hbm.at[:, pl.ds(itc, TC)], itile.at[buf], sem).wait()
                ucol = jnp.full((L,), jnp.bitwise_and(uvec[l], TC - 1), jnp.int32)
                icol = jnp.full((L,), jnp.bitwise_and(ivec[l], TC - 1), jnp.int32)
                row = (g % (NG // 2)) * L + l
                for q in range(D // L):
                    a = plsc.load_gather(utile.at[buf], [dcols[q], ucol])
                    b = plsc.load_gather(itile.at[buf], [dcols[q], icol])
                    ue2[row, pl.ds(q * L, L)] = a
                    ie2[row, pl.ds(q * L, L)] = b
            return 0

        g0 = half * (NG // 2)
        lax.fori_loop(g0, g0 + NG // 2, run_group, 0)

    def group(g, _):
        rows = (g % (NG // 2)) * L + iota
        acc = ub_v[pl.ds(g * L, L)] + ib_v[pl.ds(g * L, L)]
        for d in range(D):
            cols = jnp.full((L,), d, jnp.int32)
            a = plsc.load_gather(ue2, [rows, cols])
            b = plsc.load_gather(ie2, [rows, cols])
            acc = acc + a * b
        o_v[pl.ds(g * L, L)] = acc
        return 0

    for cp in bias_copies:
        cp.wait()

    for half in range(2):
        do_pass(half)
        g0 = half * (NG // 2)
        lax.fori_loop(g0, g0 + NG // 2, group, 0)

    pltpu.sync_copy(o_v, out_hbm.at[pl.ds(base, BW)])


def kernel(u, i, user_emb, item_emb, user_bias, item_bias):
    n_users = user_emb.shape[0]
    n_items = item_emb.shape[0]
    return _mf_sc(u.astype(jnp.int32), i.astype(jnp.int32),
                  user_emb.T, item_emb.T,
                  user_bias.T.reshape(n_users),
                  item_bias.T.reshape(n_items))


# speedup vs baseline: 1.3560x; 1.3560x over previous
import functools

import jax
import jax.numpy as jnp
from jax import lax
from jax.experimental import pallas as pl
from jax.experimental.pallas import tpu as pltpu
from jax.experimental.pallas import tpu_sc as plsc

B = 16384
D = 64
TC = 128
NC = 2
NS = 16
NW = NC * NS
BW = B // NW
L = 16
NG = BW // L
BCH = 128
NBCH = BW // BCH
HALF = BW // 2
NBUF = 2

_mesh = plsc.VectorSubcoreMesh(core_axis_name="c", subcore_axis_name="s")


@functools.partial(
    pl.kernel,
    mesh=_mesh,
    compiler_params=pltpu.CompilerParams(needs_layout_passes=False),
    out_type=jax.ShapeDtypeStruct((B,), jnp.float32),
    scratch_types=[
        pltpu.VMEM((BW,), jnp.int32),
        pltpu.VMEM((BW,), jnp.int32),
        pltpu.VMEM((NBUF, D, TC), jnp.float32),
        pltpu.VMEM((NBUF, D, TC), jnp.float32),
        pltpu.VMEM((HALF, D), jnp.float32),
        pltpu.VMEM((HALF, D), jnp.float32),
        pltpu.VMEM((BW,), jnp.float32),
        pltpu.VMEM((BW,), jnp.float32),
        pltpu.VMEM((BW,), jnp.float32),
        pltpu.SemaphoreType.DMA,
        pltpu.SemaphoreType.DMA,
        pltpu.SemaphoreType.DMA,
    ],
)
def _mf_sc(u_hbm, i_hbm, uet_hbm, iet_hbm, ub_hbm, ib_hbm, out_hbm,
           u_v, i_v, utile, itile, ue2, ie2, ub_v, ib_v, o_v,
           sem0, sem1, bsem):
    wid = lax.axis_index("s") * NC + lax.axis_index("c")
    base = wid * BW

    pltpu.sync_copy(u_hbm.at[pl.ds(base, BW)], u_v)
    pltpu.sync_copy(i_hbm.at[pl.ds(base, BW)], i_v)

    bias_copies = []
    for c in range(NBCH):
        sl = pl.ds(c * BCH, BCH)
        bias_copies.append(pltpu.async_copy(ub_hbm.at[u_v.at[sl]], ub_v.at[sl], bsem))
        bias_copies.append(pltpu.async_copy(ib_hbm.at[i_v.at[sl]], ib_v.at[sl], bsem))

    iota = lax.iota(jnp.int32, L)
    sems = (sem0, sem1)

    def tilecol(vec, l):
        return lax.shift_right_logical(vec[l], 7) * TC

    dcols = [jnp.full((L,), q * L, jnp.int32) + iota for q in range(D // L)]

    def do_pass(half):
        def run_group(g, _):
            uvec = u_v[pl.ds(g * L, L)]
            ivec = i_v[pl.ds(g * L, L)]

            def fire(l):
                buf = l % NBUF
                pltpu.async_copy(uet_hbm.at[:, pl.ds(tilecol(uvec, l), TC)],
                                 utile.at[buf], sems[buf])
                pltpu.async_copy(iet_hbm.at[:, pl.ds(tilecol(ivec, l), TC)],
                                 itile.at[buf], sems[buf])

            def extract(l):
                buf = l % NBUF
                pltpu.make_async_copy(uet_hbm.at[:, pl.ds(tilecol(uvec, l), TC)],
                                      utile.at[buf], sems[buf]).wait()
                pltpu.make_async_copy(iet_hbm.at[:, pl.ds(tilecol(ivec, l), TC)],
                                      itile.at[buf], sems[buf]).wait()
                ucol = jnp.full((L,), jnp.bitwise_and(uvec[l], TC - 1), jnp.int32)
                icol = jnp.full((L,), jnp.bitwise_and(ivec[l], TC - 1), jnp.int32)
                row = (g % (NG // 2)) * L + l
                for q in range(D // L):
                    a = plsc.load_gather(utile.at[buf], [dcols[q], ucol])
                    b = plsc.load_gather(itile.at[buf], [dcols[q], icol])
                    ue2[row, pl.ds(q * L, L)] = a
                    ie2[row, pl.ds(q * L, L)] = b

            fire(0)
            fire(1)
            for l in range(L - 2):
                extract(l)
                fire(l + 2)
            extract(L - 2)
            extract(L - 1)
            return 0

        g0 = half * (NG // 2)
        lax.fori_loop(g0, g0 + NG // 2, run_group, 0)

    def group(g, _):
        rows = (g % (NG // 2)) * L + iota
        acc = ub_v[pl.ds(g * L, L)] + ib_v[pl.ds(g * L, L)]
        for d in range(D):
            cols = jnp.full((L,), d, jnp.int32)
            a = plsc.load_gather(ue2, [rows, cols])
            b = plsc.load_gather(ie2, [rows, cols])
            acc = acc + a * b
        o_v[pl.ds(g * L, L)] = acc
        return 0

    for cp in bias_copies:
        cp.wait()

    for half in range(2):
        do_pass(half)
        g0 = half * (NG // 2)
        lax.fori_loop(g0, g0 + NG // 2, group, 0)

    pltpu.sync_copy(o_v, out_hbm.at[pl.ds(base, BW)])


def kernel(u, i, user_emb, item_emb, user_bias, item_bias):
    n_users = user_emb.shape[0]
    n_items = item_emb.shape[0]
    return _mf_sc(u.astype(jnp.int32), i.astype(jnp.int32),
                  user_emb.T, item_emb.T,
                  user_bias.T.reshape(n_users),
                  item_bias.T.reshape(n_items))


# speedup vs baseline: 1.4289x; 1.0537x over previous
import functools

import jax
import jax.numpy as jnp
from jax import lax
from jax.experimental import pallas as pl
from jax.experimental.pallas import tpu as pltpu
from jax.experimental.pallas import tpu_sc as plsc

B = 16384
D = 64
TC = 128
NC = 2
NS = 16
NW = NC * NS
BW = B // NW
L = 16
NG = BW // L
BCH = 128
NBCH = BW // BCH
HALF = BW // 2
NBUF = 3

_mesh = plsc.VectorSubcoreMesh(core_axis_name="c", subcore_axis_name="s")


@functools.partial(
    pl.kernel,
    mesh=_mesh,
    compiler_params=pltpu.CompilerParams(needs_layout_passes=False),
    out_type=jax.ShapeDtypeStruct((B,), jnp.float32),
    scratch_types=[
        pltpu.VMEM((BW,), jnp.int32),
        pltpu.VMEM((BW,), jnp.int32),
        pltpu.VMEM((NBUF, D, TC), jnp.float32),
        pltpu.VMEM((NBUF, D, TC), jnp.float32),
        pltpu.VMEM((HALF, D), jnp.float32),
        pltpu.VMEM((HALF, D), jnp.float32),
        pltpu.VMEM((BW,), jnp.float32),
        pltpu.VMEM((BW,), jnp.float32),
        pltpu.VMEM((BW,), jnp.float32),
        pltpu.SemaphoreType.DMA,
        pltpu.SemaphoreType.DMA,
        pltpu.SemaphoreType.DMA,
        pltpu.SemaphoreType.DMA,
    ],
)
def _mf_sc(u_hbm, i_hbm, uet_hbm, iet_hbm, ub_hbm, ib_hbm, out_hbm,
           u_v, i_v, utile, itile, ue2, ie2, ub_v, ib_v, o_v,
           sem0, sem1, sem2, bsem):
    wid = lax.axis_index("s") * NC + lax.axis_index("c")
    base = wid * BW

    pltpu.sync_copy(u_hbm.at[pl.ds(base, BW)], u_v)
    pltpu.sync_copy(i_hbm.at[pl.ds(base, BW)], i_v)

    bias_copies = []
    for c in range(NBCH):
        sl = pl.ds(c * BCH, BCH)
        bias_copies.append(pltpu.async_copy(ub_hbm.at[u_v.at[sl]], ub_v.at[sl], bsem))
        bias_copies.append(pltpu.async_copy(ib_hbm.at[i_v.at[sl]], ib_v.at[sl], bsem))

    iota = lax.iota(jnp.int32, L)
    sems = (sem0, sem1, sem2)

    def tilecol(vec, l):
        return lax.shift_right_logical(vec[l], 7) * TC

    dcols = [jnp.full((L,), q * L, jnp.int32) + iota for q in range(D // L)]

    def do_pass(half):
        def run_group(g, _):
            uvec = u_v[pl.ds(g * L, L)]
            ivec = i_v[pl.ds(g * L, L)]

            def fire(l):
                buf = l % NBUF
                pltpu.async_copy(uet_hbm.at[:, pl.ds(tilecol(uvec, l), TC)],
                                 utile.at[buf], sems[buf])
                pltpu.async_copy(iet_hbm.at[:, pl.ds(tilecol(ivec, l), TC)],
                                 itile.at[buf], sems[buf])

            def extract(l):
                buf = l % NBUF
                pltpu.make_async_copy(uet_hbm.at[:, pl.ds(tilecol(uvec, l), TC)],
                                      utile.at[buf], sems[buf]).wait()
                pltpu.make_async_copy(iet_hbm.at[:, pl.ds(tilecol(ivec, l), TC)],
                                      itile.at[buf], sems[buf]).wait()
                ucol = jnp.full((L,), jnp.bitwise_and(uvec[l], TC - 1), jnp.int32)
                icol = jnp.full((L,), jnp.bitwise_and(ivec[l], TC - 1), jnp.int32)
                row = (g % (NG // 2)) * L + l
                for q in range(D // L):
                    a = plsc.load_gather(utile.at[buf], [dcols[q], ucol])
                    b = plsc.load_gather(itile.at[buf], [dcols[q], icol])
                    ue2[row, pl.ds(q * L, L)] = a
                    ie2[row, pl.ds(q * L, L)] = b

            fire(0)
            fire(1)
            fire(2)
            for l in range(L - 3):
                extract(l)
                fire(l + 3)
            extract(L - 3)
            extract(L - 2)
            extract(L - 1)
            return 0

        g0 = half * (NG // 2)
        lax.fori_loop(g0, g0 + NG // 2, run_group, 0)

    def group(g, _):
        rows = (g % (NG // 2)) * L + iota
        acc = ub_v[pl.ds(g * L, L)] + ib_v[pl.ds(g * L, L)]
        for d in range(D):
            cols = jnp.full((L,), d, jnp.int32)
            a = plsc.load_gather(ue2, [rows, cols])
            b = plsc.load_gather(ie2, [rows, cols])
            acc = acc + a * b
        o_v[pl.ds(g * L, L)] = acc
        return 0

    for cp in bias_copies:
        cp.wait()

    for half in range(2):
        do_pass(half)
        g0 = half * (NG // 2)
        lax.fori_loop(g0, g0 + NG // 2, group, 0)

    pltpu.sync_copy(o_v, out_hbm.at[pl.ds(base, BW)])


def kernel(u, i, user_emb, item_emb, user_bias, item_bias):
    n_users = user_emb.shape[0]
    n_items = item_emb.shape[0]
    return _mf_sc(u.astype(jnp.int32), i.astype(jnp.int32),
                  user_emb.T, item_emb.T,
                  user_bias.T.reshape(n_users),
                  item_bias.T.reshape(n_items))
